# baseline (speedup 1.0000x reference)
import functools

import jax
import jax.numpy as jnp
from jax.experimental import pallas as pl
from jax.experimental.pallas import tpu as pltpu

BS = 16384
L = 32
NC = 10
ZD = 128
K = 128
T = 8
D = NC

CHUNK = 2048
NSTEPS = BS // CHUNK

_ROWC = (((0,), (0,)), ((), ()))


def _body(pr_ref, valid_ref, wbig_ref, bcnn_ref, mbig_ref, c2_ref, ctall_ref,
          bcomb_ref, out_ref, cmt_ref, perp_ref, counts_ref, acc_ref):
    i = pl.program_id(0)

    @pl.when(i == 0)
    def _init():
        counts_ref[...] = jnp.zeros_like(counts_ref)
        acc_ref[...] = jnp.zeros_like(acc_ref)

    pr = pr_ref[...]
    valid = valid_ref[...]

    z_all = jnp.dot(pr, wbig_ref[...], preferred_element_type=jnp.float32)
    z_all = jnp.maximum(z_all + bcnn_ref[...], 0.0)

    s_all = jnp.dot(z_all, mbig_ref[...],
                    preferred_element_type=jnp.float32) + c2_ref[...]

    iota = jax.lax.broadcasted_iota(jnp.int32, (CHUNK, K), 1)
    encs = []
    dmin_sum = jnp.zeros((CHUNK, 1), jnp.float32)
    for t in range(T):
        s_t = s_all[:, t * K:(t + 1) * K]
        dmin_sum = dmin_sum + jnp.min(s_t, axis=1, keepdims=True)
        amin = jnp.argmin(s_t, axis=1).astype(jnp.int32)
        encs.append((iota == amin[:, None]).astype(jnp.float32))
    enc_all = jnp.concatenate(encs, axis=1)

    out_ref[...] = bcomb_ref[...] + jnp.dot(
        enc_all, ctall_ref[...], preferred_element_type=jnp.float32)

    counts_ref[...] = counts_ref[...] + jax.lax.dot_general(
        valid, enc_all, _ROWC, preferred_element_type=jnp.float32)
    z2sum = jnp.sum(z_all * z_all, axis=1, keepdims=True)
    loss = jax.lax.dot_general(valid, dmin_sum + z2sum, _ROWC,
                               preferred_element_type=jnp.float32)
    vsum = jnp.sum(valid).reshape(1, 1)
    acc_ref[...] = acc_ref[...] + jnp.concatenate([loss, vsum], axis=1)

    @pl.when(i == NSTEPS - 1)
    def _fin():
        a = acc_ref[...]
        loss_sum = a[:, 0:1]
        n8 = a[:, 1:2] * T
        e_latent = loss_sum / (n8 * D + 1e-9)
        cmt_ref[...] = 0.25 * e_latent
        call = counts_ref[...]
        c128 = call[:, 0:K]
        for t in range(1, T):
            c128 = c128 + call[:, t * K:(t + 1) * K]
        p = c128 / (n8 + 1e-9)
        ent = -jnp.sum(p * jnp.log(p + 1e-10), axis=1, keepdims=True)
        perp_ref[...] = jnp.exp(ent)


@functools.partial(jax.jit, static_argnames=())
def kernel(pr, track_pad_mask, W_cnn, b_cnn, codebook, W_fc, b_fc, W_mu, b_mu):
    W_comb = W_mu @ W_fc
    b_comb = (W_mu @ b_fc + b_mu)[None, :]
    Wr = W_comb.reshape(ZD, NC, T)
    CtAll = jnp.einsum('kc,zct->tkz', codebook, Wr).reshape(T * K, ZD)
    Wc = W_cnn[:, 0, :].T
    Wbig = jnp.kron(jnp.eye(T, dtype=jnp.float32), Wc)
    bcnn = jnp.tile(b_cnn, T)[None, :]
    c2 = jnp.sum(codebook * codebook, axis=1)
    c2t = jnp.tile(c2, T)[None, :]
    Mbig = jnp.kron(jnp.eye(T, dtype=jnp.float32),
                    -2.0 * codebook.T)
    validf = 1.0 - track_pad_mask.astype(jnp.float32)

    out, cmt, perp = pl.pallas_call(
        _body,
        grid=(NSTEPS,),
        in_specs=[
            pl.BlockSpec((CHUNK, L), lambda i: (i, 0)),
            pl.BlockSpec((CHUNK, 1), lambda i: (i, 0)),
            pl.BlockSpec((L, NC * T), lambda i: (0, 0)),
            pl.BlockSpec((1, NC * T), lambda i: (0, 0)),
            pl.BlockSpec((NC * T, T * K), lambda i: (0, 0)),
            pl.BlockSpec((1, T * K), lambda i: (0, 0)),
            pl.BlockSpec((T * K, ZD), lambda i: (0, 0)),
            pl.BlockSpec((1, ZD), lambda i: (0, 0)),
        ],
        out_specs=[
            pl.BlockSpec((CHUNK, ZD), lambda i: (i, 0)),
            pl.BlockSpec((1, 1), lambda i: (0, 0)),
            pl.BlockSpec((1, 1), lambda i: (0, 0)),
        ],
        out_shape=[
            jax.ShapeDtypeStruct((BS, ZD), jnp.float32),
            jax.ShapeDtypeStruct((1, 1), jnp.float32),
            jax.ShapeDtypeStruct((1, 1), jnp.float32),
        ],
        scratch_shapes=[
            pltpu.VMEM((1, T * K), jnp.float32),
            pltpu.VMEM((1, 2), jnp.float32),
        ],
    )(pr, validf, Wbig, bcnn, Mbig, c2t, CtAll, b_comb)

    return (out, cmt[0, 0], perp[0, 0])


# speedup vs baseline: 1.1844x; 1.1844x over previous
import functools

import jax
import jax.numpy as jnp
from jax.experimental import pallas as pl
from jax.experimental.pallas import tpu as pltpu

BS = 16384
L = 32
NC = 10
ZD = 128
K = 128
T = 8
D = NC

CHUNK = 2048
NSTEPS = BS // CHUNK

_ROWC = (((0,), (0,)), ((), ()))


def _body(pr_ref, valid_ref, wbig_ref, bcnn_ref, mbig_ref, c2_ref, ctall_ref,
          bcomb_ref, out_ref, cmt_ref, perp_ref, counts_ref, acc_ref):
    i = pl.program_id(0)

    @pl.when(i == 0)
    def _init():
        counts_ref[...] = jnp.zeros_like(counts_ref)
        acc_ref[...] = jnp.zeros_like(acc_ref)

    pr = pr_ref[...]
    valid = valid_ref[...]

    z_all = jnp.dot(pr, wbig_ref[...], preferred_element_type=jnp.float32)
    z_all = jnp.maximum(z_all + bcnn_ref[...], 0.0)

    s_all = jnp.dot(z_all, mbig_ref[...],
                    preferred_element_type=jnp.float32) + c2_ref[...]

    iota = jax.lax.broadcasted_iota(jnp.int32, (CHUNK, K), 1)
    encs = []
    us = []
    for t in range(T):
        s_t = s_all[:, t * K:(t + 1) * K]
        amin = jnp.argmin(s_t, axis=1).astype(jnp.int32)
        msk = iota == amin[:, None]
        encs.append(msk.astype(jnp.bfloat16))
        us.append(jnp.where(msk, s_t, 0.0))
    enc_all = jnp.concatenate(encs, axis=1)
    u_all = jnp.concatenate(us, axis=1)

    out_ref[...] = bcomb_ref[...] + jnp.dot(
        enc_all, ctall_ref[...], preferred_element_type=jnp.float32)

    validb = valid.astype(jnp.bfloat16)
    counts_ref[...] = counts_ref[...] + jax.lax.dot_general(
        validb, enc_all, _ROWC, preferred_element_type=jnp.float32)
    z2sum = jnp.sum(z_all * z_all, axis=1, keepdims=True)
    lossvec = jax.lax.dot_general(valid, u_all, _ROWC,
                                  preferred_element_type=jnp.float32)
    loss = (jnp.sum(lossvec, axis=1, keepdims=True)
            + jax.lax.dot_general(valid, z2sum, _ROWC,
                                  preferred_element_type=jnp.float32))
    vsum = jnp.sum(valid).reshape(1, 1)
    acc_ref[...] = acc_ref[...] + jnp.concatenate([loss, vsum], axis=1)

    @pl.when(i == NSTEPS - 1)
    def _fin():
        a = acc_ref[...]
        loss_sum = a[:, 0:1]
        n8 = a[:, 1:2] * T
        e_latent = loss_sum / (n8 * D + 1e-9)
        cmt_ref[...] = 0.25 * e_latent
        call = counts_ref[...]
        c128 = call[:, 0:K]
        for t in range(1, T):
            c128 = c128 + call[:, t * K:(t + 1) * K]
        p = c128 / (n8 + 1e-9)
        ent = -jnp.sum(p * jnp.log(p + 1e-10), axis=1, keepdims=True)
        perp_ref[...] = jnp.exp(ent)


@functools.partial(jax.jit, static_argnames=())
def kernel(pr, track_pad_mask, W_cnn, b_cnn, codebook, W_fc, b_fc, W_mu, b_mu):
    W_comb = W_mu @ W_fc
    b_comb = (W_mu @ b_fc + b_mu)[None, :]
    Wr = W_comb.reshape(ZD, NC, T)
    CtAll = jnp.einsum('kc,zct->tkz', codebook, Wr).reshape(T * K, ZD)
    CtAll = CtAll.astype(jnp.bfloat16)
    Wc = W_cnn[:, 0, :].T
    Wbig = jnp.kron(jnp.eye(T, dtype=jnp.float32), Wc)
    bcnn = jnp.tile(b_cnn, T)[None, :]
    c2 = jnp.sum(codebook * codebook, axis=1)
    c2t = jnp.tile(c2, T)[None, :]
    Mbig = jnp.kron(jnp.eye(T, dtype=jnp.float32),
                    -2.0 * codebook.T)
    validf = 1.0 - track_pad_mask.astype(jnp.float32)

    out, cmt, perp = pl.pallas_call(
        _body,
        grid=(NSTEPS,),
        in_specs=[
            pl.BlockSpec((CHUNK, L), lambda i: (i, 0)),
            pl.BlockSpec((CHUNK, 1), lambda i: (i, 0)),
            pl.BlockSpec((L, NC * T), lambda i: (0, 0)),
            pl.BlockSpec((1, NC * T), lambda i: (0, 0)),
            pl.BlockSpec((NC * T, T * K), lambda i: (0, 0)),
            pl.BlockSpec((1, T * K), lambda i: (0, 0)),
            pl.BlockSpec((T * K, ZD), lambda i: (0, 0)),
            pl.BlockSpec((1, ZD), lambda i: (0, 0)),
        ],
        out_specs=[
            pl.BlockSpec((CHUNK, ZD), lambda i: (i, 0)),
            pl.BlockSpec((1, 1), lambda i: (0, 0)),
            pl.BlockSpec((1, 1), lambda i: (0, 0)),
        ],
        out_shape=[
            jax.ShapeDtypeStruct((BS, ZD), jnp.float32),
            jax.ShapeDtypeStruct((1, 1), jnp.float32),
            jax.ShapeDtypeStruct((1, 1), jnp.float32),
        ],
        scratch_shapes=[
            pltpu.VMEM((1, T * K), jnp.float32),
            pltpu.VMEM((1, 2), jnp.float32),
        ],
    )(pr, validf, Wbig, bcnn, Mbig, c2t, CtAll, b_comb)

    return (out, cmt[0, 0], perp[0, 0])
